# baseline (speedup 1.0000x reference)
import functools

import jax
import jax.numpy as jnp
from jax import lax
from jax.experimental import pallas as pl
from jax.experimental.pallas import tpu as pltpu
from jax.experimental.pallas import tpu_sc as plsc

B = 128
N = 100000
K = 15
QIDX = 38
W = 2048
NBLK = 49
NPAD = W * NBLK
BIGI = 2**31 - 1

NW = 32
CPW = 64
NG = CPW // 16


CAND = NBLK * 16
BIGF = 1e9


def _topk_body(lg_ref, nz_ref, idx_ref, mn_ref, mx_ref,
               cand_v, cand_i, mns, mxs):
    b = pl.program_id(0)

    @pl.when(b == 0)
    def _init():
        cand_v[...] = jnp.full((CAND, B), -jnp.inf, jnp.float32)
        cand_i[...] = jnp.full((CAND, B), BIGF, jnp.float32)
        mns[0, 0] = jnp.float32(jnp.inf)
        mxs[0, 0] = jnp.float32(-jnp.inf)

    x = lg_ref[...]
    u = jnp.clip(nz_ref[...], 1e-8, 1.0 - 1e-8)
    z = -jnp.log(-jnp.log(u))
    iif = lax.broadcasted_iota(jnp.int32, (W, B), 0).astype(jnp.float32)
    last = b == NBLK - 1

    @pl.when(jnp.logical_not(last))
    def _mm_full():
        mns[0, 0] = jnp.minimum(mns[0, 0], jnp.min(x))
        mxs[0, 0] = jnp.maximum(mxs[0, 0], jnp.max(x))

    @pl.when(last)
    def _mm_masked():
        ii = lax.broadcasted_iota(jnp.int32, (W, B), 0)
        valid = (b * W + ii) < N
        mns[0, 0] = jnp.minimum(
            mns[0, 0], jnp.min(jnp.where(valid, x, jnp.inf)))
        mxs[0, 0] = jnp.maximum(
            mxs[0, 0], jnp.max(jnp.where(valid, x, -jnp.inf)))

    lim = jnp.where(last, jnp.float32(N - (NBLK - 1) * W), jnp.float32(W))
    p = jnp.where(iif < lim, x + z, -jnp.inf)

    bwf = (b * W).astype(jnp.float32)
    selv, seli = [], []
    for s in range(K):
        m = jnp.max(p, axis=0, keepdims=True)
        lid = jnp.min(jnp.where(p == m, iif, BIGF), axis=0, keepdims=True)
        selv.append(m)
        seli.append(lid + bwf)
        p = jnp.where(iif == lid, -jnp.inf, p)
    bv = jnp.concatenate(
        selv + [jnp.full((1, B), -jnp.inf, jnp.float32)], axis=0)
    bi = jnp.concatenate(seli + [jnp.full((1, B), BIGF, jnp.float32)], axis=0)
    cand_v[pl.ds(b * 16, 16), :] = bv
    cand_i[pl.ds(b * 16, 16), :] = bi

    @pl.when(last)
    def _fin():
        v = cand_v[...]
        iid = cand_i[...]
        sel2 = []
        for s in range(K):
            m = jnp.max(v, axis=0, keepdims=True)
            sid = jnp.min(jnp.where(v == m, iid, BIGF), axis=0, keepdims=True)
            sel2.append(sid)
            v = jnp.where((v == m) & (iid == sid), -jnp.inf, v)
        ids = jnp.concatenate(
            sel2 + [jnp.zeros((1, B), jnp.float32)], axis=0)
        idx_ref[...] = ids.astype(jnp.int32)
        mn_ref[0, 0] = mns[0, 0]
        mx_ref[0, 0] = mxs[0, 0]


def _topk_call(lgT, nzT):
    return pl.pallas_call(
        _topk_body,
        grid=(NBLK,),
        in_specs=[
            pl.BlockSpec((W, B), lambda b: (b, 0)),
            pl.BlockSpec((W, B), lambda b: (b, 0)),
        ],
        out_specs=[
            pl.BlockSpec((16, B), lambda b: (0, 0)),
            pl.BlockSpec(memory_space=pltpu.SMEM),
            pl.BlockSpec(memory_space=pltpu.SMEM),
        ],
        out_shape=[
            jax.ShapeDtypeStruct((16, B), jnp.int32),
            jax.ShapeDtypeStruct((1, 1), jnp.float32),
            jax.ShapeDtypeStruct((1, 1), jnp.float32),
        ],
        scratch_shapes=[
            pltpu.VMEM((CAND, B), jnp.float32),
            pltpu.VMEM((CAND, B), jnp.float32),
            pltpu.SMEM((1, 1), jnp.float32),
            pltpu.SMEM((1, 1), jnp.float32),
        ],
        compiler_params=pltpu.CompilerParams(
            dimension_semantics=("arbitrary",)),
    )(lgT, nzT)


def _sc_body(tab_hbm, idxp_hbm, out_hbm, cols_v, gdat_v, sem):
    c = lax.axis_index("c")
    s = lax.axis_index("s")
    wid = s * 2 + c

    pltpu.sync_copy(idxp_hbm.at[wid], cols_v)
    cp = pltpu.make_async_copy(tab_hbm.at[cols_v], gdat_v, sem)
    cp.start()
    cp.wait()
    pltpu.sync_copy(gdat_v, out_hbm.at[pl.ds(wid * CPW, CPW)])


def _sc_call(table, idx_pad):
    mesh = plsc.VectorSubcoreMesh(core_axis_name="c", subcore_axis_name="s")
    fn = functools.partial(
        pl.kernel,
        out_type=jax.ShapeDtypeStruct((NW * CPW, B), jnp.float32),
        mesh=mesh,
        scratch_types=[
            pltpu.VMEM((CPW,), jnp.int32),
            pltpu.VMEM((CPW, B), jnp.float32),
            pltpu.SemaphoreType.DMA,
        ],
    )(_sc_body)
    return fn(table, idx_pad)


TPAD = NW * CPW


def _att_body(g_ref, mn_ref, mx_ref, out_ref):
    x = g_ref[...]
    mn = mn_ref[0, 0]
    mx = mx_ref[0, 0]
    a = (x - mn) / mx
    rows = lax.broadcasted_iota(jnp.int32, (TPAD, B), 0)
    cols = lax.broadcasted_iota(jnp.int32, (TPAD, B), 1)
    imap = jnp.minimum(rows // K, B - 1)
    sel = (cols == imap).astype(jnp.float32)
    ai = jnp.sum(a * sel, axis=1, keepdims=True)
    cnt = jnp.sum((a <= ai).astype(jnp.int32), axis=1, keepdims=True)
    out_ref[...] = jnp.where(cnt <= QIDX, 0.0, ai)


def _att_call(g, mn, mx):
    return pl.pallas_call(
        _att_body,
        in_specs=[
            pl.BlockSpec((TPAD, B), lambda: (0, 0)),
            pl.BlockSpec(memory_space=pltpu.SMEM),
            pl.BlockSpec(memory_space=pltpu.SMEM),
        ],
        out_specs=pl.BlockSpec((TPAD, 1), lambda: (0, 0)),
        out_shape=jax.ShapeDtypeStruct((TPAD, 1), jnp.float32),
    )(g, mn, mx)


def kernel(logits, noise, k):
    del k
    lgT = logits.T
    nzT = noise.T
    idx16, mn, mx = _topk_call(lgT, nzT)
    idxk = idx16[:K].T

    flat_idx = idxk.reshape(-1)
    idx_pad = jnp.pad(flat_idx, (0, TPAD - B * K)).reshape(NW, CPW)
    g = _sc_call(lgT, idx_pad)
    att2 = _att_call(g, mn, mx)
    att = att2.reshape(-1)[:B * K].reshape(B, K)
    return att, idxk


# speedup vs baseline: 1.9158x; 1.9158x over previous
import functools

import jax
import jax.numpy as jnp
from jax import lax
from jax.experimental import pallas as pl
from jax.experimental.pallas import tpu as pltpu
from jax.experimental.pallas import tpu_sc as plsc

B = 128
N = 100000
K = 15
QIDX = 38
W = 2048
NBLK = 49
NPAD = W * NBLK
BIGI = 2**31 - 1

NW = 32
CPW = 64
NG = CPW // 16


CAND = NBLK * 128
BIGF = 1e9


def _topk_body(lg_ref, nz_ref, idx_ref, mn_ref, mx_ref,
               cand_v, cand_i, mns, mxs):
    b = pl.program_id(0)

    @pl.when(b == 0)
    def _init():
        cand_v[...] = jnp.full((B, CAND), -jnp.inf, jnp.float32)
        cand_i[...] = jnp.full((B, CAND), BIGF, jnp.float32)
        mns[0, 0] = jnp.float32(jnp.inf)
        mxs[0, 0] = jnp.float32(-jnp.inf)

    xt = lg_ref[...]
    u = jnp.clip(nz_ref[...], 1e-8, 1.0 - 1e-8)
    z = -jnp.log(-jnp.log(u))
    last = b == NBLK - 1

    @pl.when(jnp.logical_not(last))
    def _mm_full():
        mns[0, 0] = jnp.minimum(mns[0, 0], jnp.min(xt))
        mxs[0, 0] = jnp.maximum(mxs[0, 0], jnp.max(xt))

    @pl.when(last)
    def _mm_masked():
        ii = lax.broadcasted_iota(jnp.int32, (W, B), 0)
        valid = (b * W + ii) < N
        mns[0, 0] = jnp.minimum(
            mns[0, 0], jnp.min(jnp.where(valid, xt, jnp.inf)))
        mxs[0, 0] = jnp.maximum(
            mxs[0, 0], jnp.max(jnp.where(valid, xt, -jnp.inf)))

    iif = lax.broadcasted_iota(jnp.int32, (B, W), 1).astype(jnp.float32)
    lim = jnp.where(last, jnp.float32(N - (NBLK - 1) * W), jnp.float32(W))
    p = jnp.where(iif < lim, (xt + z).T, -jnp.inf)

    bwf = (b * W).astype(jnp.float32)
    selv, seli = [], []
    for s in range(K):
        m = jnp.max(p, axis=1, keepdims=True)
        lid = jnp.min(jnp.where(p == m, iif, BIGF), axis=1, keepdims=True)
        selv.append(m)
        seli.append(lid + bwf)
        p = jnp.where(iif == lid, -jnp.inf, p)
    bv = jnp.concatenate(
        selv + [jnp.full((B, 128 - K), -jnp.inf, jnp.float32)], axis=1)
    bi = jnp.concatenate(
        seli + [jnp.full((B, 128 - K), BIGF, jnp.float32)], axis=1)
    cand_v[:, pl.ds(b * 128, 128)] = bv
    cand_i[:, pl.ds(b * 128, 128)] = bi

    @pl.when(last)
    def _fin():
        v = cand_v[...]
        iid = cand_i[...]
        sel2 = []
        for s in range(K):
            m = jnp.max(v, axis=1, keepdims=True)
            sid = jnp.min(jnp.where(v == m, iid, BIGF), axis=1, keepdims=True)
            sel2.append(sid)
            v = jnp.where((v == m) & (iid == sid), -jnp.inf, v)
        ids = jnp.concatenate(
            sel2 + [jnp.zeros((B, 1), jnp.float32)], axis=1)
        idx_ref[...] = ids.astype(jnp.int32)
        mn_ref[0, 0] = mns[0, 0]
        mx_ref[0, 0] = mxs[0, 0]


def _topk_call(lgT, nzT):
    return pl.pallas_call(
        _topk_body,
        grid=(NBLK,),
        in_specs=[
            pl.BlockSpec((W, B), lambda b: (b, 0)),
            pl.BlockSpec((W, B), lambda b: (b, 0)),
        ],
        out_specs=[
            pl.BlockSpec((B, 16), lambda b: (0, 0)),
            pl.BlockSpec(memory_space=pltpu.SMEM),
            pl.BlockSpec(memory_space=pltpu.SMEM),
        ],
        out_shape=[
            jax.ShapeDtypeStruct((B, 16), jnp.int32),
            jax.ShapeDtypeStruct((1, 1), jnp.float32),
            jax.ShapeDtypeStruct((1, 1), jnp.float32),
        ],
        scratch_shapes=[
            pltpu.VMEM((B, CAND), jnp.float32),
            pltpu.VMEM((B, CAND), jnp.float32),
            pltpu.SMEM((1, 1), jnp.float32),
            pltpu.SMEM((1, 1), jnp.float32),
        ],
        compiler_params=pltpu.CompilerParams(
            dimension_semantics=("arbitrary",)),
    )(lgT, nzT)


def _sc_body(tab_hbm, idxp_hbm, out_hbm, cols_v, gdat_v, sem):
    c = lax.axis_index("c")
    s = lax.axis_index("s")
    wid = s * 2 + c

    pltpu.sync_copy(idxp_hbm.at[wid], cols_v)
    cp = pltpu.make_async_copy(tab_hbm.at[cols_v], gdat_v, sem)
    cp.start()
    cp.wait()
    pltpu.sync_copy(gdat_v, out_hbm.at[pl.ds(wid * CPW, CPW)])


def _sc_call(table, idx_pad):
    mesh = plsc.VectorSubcoreMesh(core_axis_name="c", subcore_axis_name="s")
    fn = functools.partial(
        pl.kernel,
        out_type=jax.ShapeDtypeStruct((NW * CPW, B), jnp.float32),
        mesh=mesh,
        scratch_types=[
            pltpu.VMEM((CPW,), jnp.int32),
            pltpu.VMEM((CPW, B), jnp.float32),
            pltpu.SemaphoreType.DMA,
        ],
    )(_sc_body)
    return fn(table, idx_pad)


TPAD = NW * CPW


def _att_body(g_ref, mn_ref, mx_ref, out_ref):
    x = g_ref[...]
    mn = mn_ref[0, 0]
    mx = mx_ref[0, 0]
    a = (x - mn) / mx
    rows = lax.broadcasted_iota(jnp.int32, (TPAD, B), 0)
    cols = lax.broadcasted_iota(jnp.int32, (TPAD, B), 1)
    imap = jnp.minimum(rows // K, B - 1)
    sel = (cols == imap).astype(jnp.float32)
    ai = jnp.sum(a * sel, axis=1, keepdims=True)
    cnt = jnp.sum((a <= ai).astype(jnp.int32), axis=1, keepdims=True)
    out_ref[...] = jnp.where(cnt <= QIDX, 0.0, ai)


def _att_call(g, mn, mx):
    return pl.pallas_call(
        _att_body,
        in_specs=[
            pl.BlockSpec((TPAD, B), lambda: (0, 0)),
            pl.BlockSpec(memory_space=pltpu.SMEM),
            pl.BlockSpec(memory_space=pltpu.SMEM),
        ],
        out_specs=pl.BlockSpec((TPAD, 1), lambda: (0, 0)),
        out_shape=jax.ShapeDtypeStruct((TPAD, 1), jnp.float32),
    )(g, mn, mx)


def kernel(logits, noise, k):
    del k
    lgT = logits.T
    nzT = noise.T
    idx16, mn, mx = _topk_call(lgT, nzT)
    idxk = idx16[:, :K]

    flat_idx = idxk.reshape(-1)
    idx_pad = jnp.pad(flat_idx, (0, TPAD - B * K)).reshape(NW, CPW)
    g = _sc_call(lgT, idx_pad)
    att2 = _att_call(g, mn, mx)
    att = att2.reshape(-1)[:B * K].reshape(B, K)
    return att, idxk


# speedup vs baseline: 1.9780x; 1.0324x over previous
import functools

import jax
import jax.numpy as jnp
from jax import lax
from jax.experimental import pallas as pl
from jax.experimental.pallas import tpu as pltpu
from jax.experimental.pallas import tpu_sc as plsc

B = 128
N = 100000
K = 15
QIDX = 38
W = 2048
NBLK = 49
NPAD = W * NBLK
BIGI = 2**31 - 1

NW = 32
CPW = 64
NG = CPW // 16


CAND = NBLK * 128
BIGF = 1e9


def _topk_body(lg_ref, nz_ref, idx_ref, mn_ref, mx_ref,
               cand_v, cand_i, mns, mxs):
    b = pl.program_id(0)

    @pl.when(b == 0)
    def _init():
        cand_v[...] = jnp.full((B, CAND), -jnp.inf, jnp.float32)
        cand_i[...] = jnp.full((B, CAND), BIGF, jnp.float32)
        mns[0, 0] = jnp.float32(jnp.inf)
        mxs[0, 0] = jnp.float32(-jnp.inf)

    xt = lg_ref[...]
    u = jnp.clip(nz_ref[...], 1e-8, 1.0 - 1e-8)
    z = -jnp.log(-jnp.log(u))
    last = b == NBLK - 1

    @pl.when(jnp.logical_not(last))
    def _mm_full():
        mns[0, 0] = jnp.minimum(mns[0, 0], jnp.min(xt))
        mxs[0, 0] = jnp.maximum(mxs[0, 0], jnp.max(xt))

    @pl.when(last)
    def _mm_masked():
        ii = lax.broadcasted_iota(jnp.int32, (W, B), 0)
        valid = (b * W + ii) < N
        mns[0, 0] = jnp.minimum(
            mns[0, 0], jnp.min(jnp.where(valid, xt, jnp.inf)))
        mxs[0, 0] = jnp.maximum(
            mxs[0, 0], jnp.max(jnp.where(valid, xt, -jnp.inf)))

    iif = lax.broadcasted_iota(jnp.int32, (B, W), 1).astype(jnp.float32)
    lim = jnp.where(last, jnp.float32(N - (NBLK - 1) * W), jnp.float32(W))
    p = jnp.where(iif < lim, (xt + z).T, -jnp.inf)

    bwf = (b * W).astype(jnp.float32)
    selv, seli = [], []
    for s in range(K):
        m = jnp.max(p, axis=1, keepdims=True)
        lid = jnp.min(jnp.where(p == m, iif, BIGF), axis=1, keepdims=True)
        selv.append(m)
        seli.append(lid + bwf)
        p = jnp.where(iif == lid, -jnp.inf, p)
    bv = jnp.concatenate(
        selv + [jnp.full((B, 128 - K), -jnp.inf, jnp.float32)], axis=1)
    bi = jnp.concatenate(
        seli + [jnp.full((B, 128 - K), BIGF, jnp.float32)], axis=1)
    cand_v[:, pl.ds(b * 128, 128)] = bv
    cand_i[:, pl.ds(b * 128, 128)] = bi

    @pl.when(last)
    def _fin():
        v = jnp.concatenate(
            [cand_v[:, j * 128:j * 128 + 16] for j in range(NBLK)]
            + [jnp.full((B, 112), -jnp.inf, jnp.float32)], axis=1)
        iid = jnp.concatenate(
            [cand_i[:, j * 128:j * 128 + 16] for j in range(NBLK)]
            + [jnp.full((B, 112), BIGF, jnp.float32)], axis=1)
        sel2 = []
        for s in range(K):
            m = jnp.max(v, axis=1, keepdims=True)
            sid = jnp.min(jnp.where(v == m, iid, BIGF), axis=1, keepdims=True)
            sel2.append(sid)
            v = jnp.where((v == m) & (iid == sid), -jnp.inf, v)
        ids = jnp.concatenate(
            sel2 + [jnp.zeros((B, 1), jnp.float32)], axis=1)
        idx_ref[...] = ids.astype(jnp.int32)
        mn_ref[0, 0] = mns[0, 0]
        mx_ref[0, 0] = mxs[0, 0]


def _topk_call(lgT, nzT):
    return pl.pallas_call(
        _topk_body,
        grid=(NBLK,),
        in_specs=[
            pl.BlockSpec((W, B), lambda b: (b, 0)),
            pl.BlockSpec((W, B), lambda b: (b, 0)),
        ],
        out_specs=[
            pl.BlockSpec((B, 16), lambda b: (0, 0)),
            pl.BlockSpec(memory_space=pltpu.SMEM),
            pl.BlockSpec(memory_space=pltpu.SMEM),
        ],
        out_shape=[
            jax.ShapeDtypeStruct((B, 16), jnp.int32),
            jax.ShapeDtypeStruct((1, 1), jnp.float32),
            jax.ShapeDtypeStruct((1, 1), jnp.float32),
        ],
        scratch_shapes=[
            pltpu.VMEM((B, CAND), jnp.float32),
            pltpu.VMEM((B, CAND), jnp.float32),
            pltpu.SMEM((1, 1), jnp.float32),
            pltpu.SMEM((1, 1), jnp.float32),
        ],
        compiler_params=pltpu.CompilerParams(
            dimension_semantics=("arbitrary",)),
    )(lgT, nzT)


def _sc_body(tab_hbm, idxp_hbm, out_hbm, cols_v, gdat_v, sem):
    c = lax.axis_index("c")
    s = lax.axis_index("s")
    wid = s * 2 + c

    pltpu.sync_copy(idxp_hbm.at[wid], cols_v)
    cp = pltpu.make_async_copy(tab_hbm.at[cols_v], gdat_v, sem)
    cp.start()
    cp.wait()
    pltpu.sync_copy(gdat_v, out_hbm.at[pl.ds(wid * CPW, CPW)])


def _sc_call(table, idx_pad):
    mesh = plsc.VectorSubcoreMesh(core_axis_name="c", subcore_axis_name="s")
    fn = functools.partial(
        pl.kernel,
        out_type=jax.ShapeDtypeStruct((NW * CPW, B), jnp.float32),
        mesh=mesh,
        scratch_types=[
            pltpu.VMEM((CPW,), jnp.int32),
            pltpu.VMEM((CPW, B), jnp.float32),
            pltpu.SemaphoreType.DMA,
        ],
    )(_sc_body)
    return fn(table, idx_pad)


TPAD = NW * CPW


def _att_body(g_ref, mn_ref, mx_ref, out_ref):
    x = g_ref[...]
    mn = mn_ref[0, 0]
    mx = mx_ref[0, 0]
    a = (x - mn) / mx
    rows = lax.broadcasted_iota(jnp.int32, (TPAD, B), 0)
    cols = lax.broadcasted_iota(jnp.int32, (TPAD, B), 1)
    imap = jnp.minimum(rows // K, B - 1)
    sel = (cols == imap).astype(jnp.float32)
    ai = jnp.sum(a * sel, axis=1, keepdims=True)
    cnt = jnp.sum((a <= ai).astype(jnp.int32), axis=1, keepdims=True)
    out_ref[...] = jnp.where(cnt <= QIDX, 0.0, ai)


def _att_call(g, mn, mx):
    return pl.pallas_call(
        _att_body,
        in_specs=[
            pl.BlockSpec((TPAD, B), lambda: (0, 0)),
            pl.BlockSpec(memory_space=pltpu.SMEM),
            pl.BlockSpec(memory_space=pltpu.SMEM),
        ],
        out_specs=pl.BlockSpec((TPAD, 1), lambda: (0, 0)),
        out_shape=jax.ShapeDtypeStruct((TPAD, 1), jnp.float32),
    )(g, mn, mx)


def kernel(logits, noise, k):
    del k
    lgT = logits.T
    nzT = noise.T
    idx16, mn, mx = _topk_call(lgT, nzT)
    idxk = idx16[:, :K]

    flat_idx = idxk.reshape(-1)
    idx_pad = jnp.pad(flat_idx, (0, TPAD - B * K)).reshape(NW, CPW)
    g = _sc_call(lgT, idx_pad)
    att2 = _att_call(g, mn, mx)
    att = att2.reshape(-1)[:B * K].reshape(B, K)
    return att, idxk


# speedup vs baseline: 1.9859x; 1.0040x over previous
import functools

import jax
import jax.numpy as jnp
from jax import lax
from jax.experimental import pallas as pl
from jax.experimental.pallas import tpu as pltpu
from jax.experimental.pallas import tpu_sc as plsc

B = 128
N = 100000
K = 15
QIDX = 38
W = 4096
NBLK = 25
NPAD = W * NBLK
BIGI = 2**31 - 1

NW = 32
CPW = 64
NG = CPW // 16


CAND = NBLK * 128
BIGF = 1e9


def _topk_body(lg_ref, nz_ref, idx_ref, mn_ref, mx_ref,
               cand_v, cand_i, mns, mxs):
    b = pl.program_id(0)

    @pl.when(b == 0)
    def _init():
        cand_v[...] = jnp.full((B, CAND), -jnp.inf, jnp.float32)
        cand_i[...] = jnp.full((B, CAND), BIGF, jnp.float32)
        mns[0, 0] = jnp.float32(jnp.inf)
        mxs[0, 0] = jnp.float32(-jnp.inf)

    xt = lg_ref[...]
    u = jnp.clip(nz_ref[...], 1e-8, 1.0 - 1e-8)
    z = -jnp.log(-jnp.log(u))
    last = b == NBLK - 1

    @pl.when(jnp.logical_not(last))
    def _mm_full():
        mns[0, 0] = jnp.minimum(mns[0, 0], jnp.min(xt))
        mxs[0, 0] = jnp.maximum(mxs[0, 0], jnp.max(xt))

    @pl.when(last)
    def _mm_masked():
        ii = lax.broadcasted_iota(jnp.int32, (W, B), 0)
        valid = (b * W + ii) < N
        mns[0, 0] = jnp.minimum(
            mns[0, 0], jnp.min(jnp.where(valid, xt, jnp.inf)))
        mxs[0, 0] = jnp.maximum(
            mxs[0, 0], jnp.max(jnp.where(valid, xt, -jnp.inf)))

    iif = lax.broadcasted_iota(jnp.int32, (B, W), 1).astype(jnp.float32)
    lim = jnp.where(last, jnp.float32(N - (NBLK - 1) * W), jnp.float32(W))
    p = jnp.where(iif < lim, (xt + z).T, -jnp.inf)

    bwf = (b * W).astype(jnp.float32)
    selv, seli = [], []
    for s in range(K):
        m = jnp.max(p, axis=1, keepdims=True)
        lid = jnp.min(jnp.where(p == m, iif, BIGF), axis=1, keepdims=True)
        selv.append(m)
        seli.append(lid + bwf)
        p = jnp.where(iif == lid, -jnp.inf, p)
    bv = jnp.concatenate(
        selv + [jnp.full((B, 128 - K), -jnp.inf, jnp.float32)], axis=1)
    bi = jnp.concatenate(
        seli + [jnp.full((B, 128 - K), BIGF, jnp.float32)], axis=1)
    cand_v[:, pl.ds(b * 128, 128)] = bv
    cand_i[:, pl.ds(b * 128, 128)] = bi

    @pl.when(last)
    def _fin():
        v = jnp.concatenate(
            [cand_v[:, j * 128:j * 128 + 16] for j in range(NBLK)]
            + [jnp.full((B, 112), -jnp.inf, jnp.float32)], axis=1)
        iid = jnp.concatenate(
            [cand_i[:, j * 128:j * 128 + 16] for j in range(NBLK)]
            + [jnp.full((B, 112), BIGF, jnp.float32)], axis=1)
        sel2 = []
        for s in range(K):
            m = jnp.max(v, axis=1, keepdims=True)
            sid = jnp.min(jnp.where(v == m, iid, BIGF), axis=1, keepdims=True)
            sel2.append(sid)
            v = jnp.where((v == m) & (iid == sid), -jnp.inf, v)
        ids = jnp.concatenate(
            sel2 + [jnp.zeros((B, 1), jnp.float32)], axis=1)
        idx_ref[...] = ids.astype(jnp.int32)
        mn_ref[0, 0] = mns[0, 0]
        mx_ref[0, 0] = mxs[0, 0]


def _topk_call(lgT, nzT):
    return pl.pallas_call(
        _topk_body,
        grid=(NBLK,),
        in_specs=[
            pl.BlockSpec((W, B), lambda b: (b, 0)),
            pl.BlockSpec((W, B), lambda b: (b, 0)),
        ],
        out_specs=[
            pl.BlockSpec((B, 16), lambda b: (0, 0)),
            pl.BlockSpec(memory_space=pltpu.SMEM),
            pl.BlockSpec(memory_space=pltpu.SMEM),
        ],
        out_shape=[
            jax.ShapeDtypeStruct((B, 16), jnp.int32),
            jax.ShapeDtypeStruct((1, 1), jnp.float32),
            jax.ShapeDtypeStruct((1, 1), jnp.float32),
        ],
        scratch_shapes=[
            pltpu.VMEM((B, CAND), jnp.float32),
            pltpu.VMEM((B, CAND), jnp.float32),
            pltpu.SMEM((1, 1), jnp.float32),
            pltpu.SMEM((1, 1), jnp.float32),
        ],
        compiler_params=pltpu.CompilerParams(
            dimension_semantics=("arbitrary",)),
    )(lgT, nzT)


def _sc_body(tab_hbm, idxp_hbm, out_hbm, cols_v, gdat_v, sem):
    c = lax.axis_index("c")
    s = lax.axis_index("s")
    wid = s * 2 + c

    pltpu.sync_copy(idxp_hbm.at[wid], cols_v)
    cp = pltpu.make_async_copy(tab_hbm.at[cols_v], gdat_v, sem)
    cp.start()
    cp.wait()
    pltpu.sync_copy(gdat_v, out_hbm.at[pl.ds(wid * CPW, CPW)])


def _sc_call(table, idx_pad):
    mesh = plsc.VectorSubcoreMesh(core_axis_name="c", subcore_axis_name="s")
    fn = functools.partial(
        pl.kernel,
        out_type=jax.ShapeDtypeStruct((NW * CPW, B), jnp.float32),
        mesh=mesh,
        scratch_types=[
            pltpu.VMEM((CPW,), jnp.int32),
            pltpu.VMEM((CPW, B), jnp.float32),
            pltpu.SemaphoreType.DMA,
        ],
    )(_sc_body)
    return fn(table, idx_pad)


TPAD = NW * CPW


def _att_body(g_ref, mn_ref, mx_ref, out_ref):
    x = g_ref[...]
    mn = mn_ref[0, 0]
    mx = mx_ref[0, 0]
    a = (x - mn) / mx
    rows = lax.broadcasted_iota(jnp.int32, (TPAD, B), 0)
    cols = lax.broadcasted_iota(jnp.int32, (TPAD, B), 1)
    imap = jnp.minimum(rows // K, B - 1)
    sel = (cols == imap).astype(jnp.float32)
    ai = jnp.sum(a * sel, axis=1, keepdims=True)
    cnt = jnp.sum((a <= ai).astype(jnp.int32), axis=1, keepdims=True)
    out_ref[...] = jnp.where(cnt <= QIDX, 0.0, ai)


def _att_call(g, mn, mx):
    return pl.pallas_call(
        _att_body,
        in_specs=[
            pl.BlockSpec((TPAD, B), lambda: (0, 0)),
            pl.BlockSpec(memory_space=pltpu.SMEM),
            pl.BlockSpec(memory_space=pltpu.SMEM),
        ],
        out_specs=pl.BlockSpec((TPAD, 1), lambda: (0, 0)),
        out_shape=jax.ShapeDtypeStruct((TPAD, 1), jnp.float32),
    )(g, mn, mx)


def kernel(logits, noise, k):
    del k
    lgT = logits.T
    nzT = noise.T
    idx16, mn, mx = _topk_call(lgT, nzT)
    idxk = idx16[:, :K]

    flat_idx = idxk.reshape(-1)
    idx_pad = jnp.pad(flat_idx, (0, TPAD - B * K)).reshape(NW, CPW)
    g = _sc_call(lgT, idx_pad)
    att2 = _att_call(g, mn, mx)
    att = att2.reshape(-1)[:B * K].reshape(B, K)
    return att, idxk


# speedup vs baseline: 1.9861x; 1.0001x over previous
import functools

import jax
import jax.numpy as jnp
from jax import lax
from jax.experimental import pallas as pl
from jax.experimental.pallas import tpu as pltpu
from jax.experimental.pallas import tpu_sc as plsc

B = 128
N = 100000
K = 15
QIDX = 38
W = 4096
NBLK = 25

NW = 32
CPW = 64


CAND = NBLK * 128
BIGF = 1e9


def _topk_body(lg_ref, nz_ref, idx_ref, mn_ref, mx_ref,
               cand_v, cand_i, mns, mxs):
    b = pl.program_id(0)

    @pl.when(b == 0)
    def _init():
        cand_v[...] = jnp.full((B, CAND), -jnp.inf, jnp.float32)
        cand_i[...] = jnp.full((B, CAND), BIGF, jnp.float32)
        mns[0, 0] = jnp.float32(jnp.inf)
        mxs[0, 0] = jnp.float32(-jnp.inf)

    xt = lg_ref[...]
    u = jnp.clip(nz_ref[...], 1e-8, 1.0 - 1e-8)
    z = -jnp.log(-jnp.log(u))
    last = b == NBLK - 1

    @pl.when(jnp.logical_not(last))
    def _mm_full():
        mns[0, 0] = jnp.minimum(mns[0, 0], jnp.min(xt))
        mxs[0, 0] = jnp.maximum(mxs[0, 0], jnp.max(xt))

    @pl.when(last)
    def _mm_masked():
        ii = lax.broadcasted_iota(jnp.int32, (W, B), 0)
        valid = (b * W + ii) < N
        mns[0, 0] = jnp.minimum(
            mns[0, 0], jnp.min(jnp.where(valid, xt, jnp.inf)))
        mxs[0, 0] = jnp.maximum(
            mxs[0, 0], jnp.max(jnp.where(valid, xt, -jnp.inf)))

    iif = lax.broadcasted_iota(jnp.int32, (B, W), 1).astype(jnp.float32)
    lim = jnp.where(last, jnp.float32(N - (NBLK - 1) * W), jnp.float32(W))
    p = jnp.where(iif < lim, (xt + z).T, -jnp.inf)

    bwf = (b * W).astype(jnp.float32)
    selv, seli = [], []
    for s in range(K):
        m = jnp.max(p, axis=1, keepdims=True)
        lid = jnp.min(jnp.where(p == m, iif, BIGF), axis=1, keepdims=True)
        selv.append(m)
        seli.append(lid + bwf)
        p = jnp.where(iif == lid, -jnp.inf, p)
    bv = jnp.concatenate(
        selv + [jnp.full((B, 128 - K), -jnp.inf, jnp.float32)], axis=1)
    bi = jnp.concatenate(
        seli + [jnp.full((B, 128 - K), BIGF, jnp.float32)], axis=1)
    cand_v[:, pl.ds(b * 128, 128)] = bv
    cand_i[:, pl.ds(b * 128, 128)] = bi

    @pl.when(last)
    def _fin():
        v = jnp.concatenate(
            [cand_v[:, j * 128:j * 128 + 16] for j in range(NBLK)]
            + [jnp.full((B, 112), -jnp.inf, jnp.float32)], axis=1)
        iid = jnp.concatenate(
            [cand_i[:, j * 128:j * 128 + 16] for j in range(NBLK)]
            + [jnp.full((B, 112), BIGF, jnp.float32)], axis=1)
        sel2 = []
        for s in range(K):
            m = jnp.max(v, axis=1, keepdims=True)
            sid = jnp.min(jnp.where(v == m, iid, BIGF), axis=1, keepdims=True)
            sel2.append(sid)
            v = jnp.where((v == m) & (iid == sid), -jnp.inf, v)
        ids = jnp.concatenate(
            sel2 + [jnp.zeros((B, 1), jnp.float32)], axis=1)
        idx_ref[...] = ids.astype(jnp.int32)
        mn_ref[0, 0] = mns[0, 0]
        mx_ref[0, 0] = mxs[0, 0]


def _topk_call(lgT, nzT):
    return pl.pallas_call(
        _topk_body,
        grid=(NBLK,),
        in_specs=[
            pl.BlockSpec((W, B), lambda b: (b, 0)),
            pl.BlockSpec((W, B), lambda b: (b, 0)),
        ],
        out_specs=[
            pl.BlockSpec((B, 16), lambda b: (0, 0)),
            pl.BlockSpec(memory_space=pltpu.SMEM),
            pl.BlockSpec(memory_space=pltpu.SMEM),
        ],
        out_shape=[
            jax.ShapeDtypeStruct((B, 16), jnp.int32),
            jax.ShapeDtypeStruct((1, 1), jnp.float32),
            jax.ShapeDtypeStruct((1, 1), jnp.float32),
        ],
        scratch_shapes=[
            pltpu.VMEM((B, CAND), jnp.float32),
            pltpu.VMEM((B, CAND), jnp.float32),
            pltpu.SMEM((1, 1), jnp.float32),
            pltpu.SMEM((1, 1), jnp.float32),
        ],
        compiler_params=pltpu.CompilerParams(
            dimension_semantics=("arbitrary",)),
    )(lgT, nzT)


def _sc_body(tab_hbm, idxp_hbm, out_hbm, cols_v, gdat_v, sem):
    c = lax.axis_index("c")
    s = lax.axis_index("s")
    wid = s * 2 + c

    pltpu.sync_copy(idxp_hbm.at[wid], cols_v)
    cp = pltpu.make_async_copy(tab_hbm.at[cols_v], gdat_v, sem)
    cp.start()
    cp.wait()
    pltpu.sync_copy(gdat_v, out_hbm.at[pl.ds(wid * CPW, CPW)])


def _sc_call(table, idx_pad):
    mesh = plsc.VectorSubcoreMesh(core_axis_name="c", subcore_axis_name="s")
    fn = functools.partial(
        pl.kernel,
        out_type=jax.ShapeDtypeStruct((NW * CPW, B), jnp.float32),
        mesh=mesh,
        scratch_types=[
            pltpu.VMEM((CPW,), jnp.int32),
            pltpu.VMEM((CPW, B), jnp.float32),
            pltpu.SemaphoreType.DMA,
        ],
    )(_sc_body)
    return fn(table, idx_pad)


TPAD = NW * CPW


def _att_body(g_ref, mn_ref, mx_ref, out_ref):
    x = g_ref[...]
    mn = mn_ref[0, 0]
    mx = mx_ref[0, 0]
    a = (x - mn) / mx
    rows = lax.broadcasted_iota(jnp.int32, (TPAD, B), 0)
    cols = lax.broadcasted_iota(jnp.int32, (TPAD, B), 1)
    imap = jnp.minimum(rows // K, B - 1)
    sel = (cols == imap).astype(jnp.float32)
    ai = jnp.sum(a * sel, axis=1, keepdims=True)
    cnt = jnp.sum((a <= ai).astype(jnp.int32), axis=1, keepdims=True)
    out_ref[...] = jnp.where(cnt <= QIDX, 0.0, ai)


def _att_call(g, mn, mx):
    return pl.pallas_call(
        _att_body,
        in_specs=[
            pl.BlockSpec((TPAD, B), lambda: (0, 0)),
            pl.BlockSpec(memory_space=pltpu.SMEM),
            pl.BlockSpec(memory_space=pltpu.SMEM),
        ],
        out_specs=pl.BlockSpec((TPAD, 1), lambda: (0, 0)),
        out_shape=jax.ShapeDtypeStruct((TPAD, 1), jnp.float32),
    )(g, mn, mx)


def kernel(logits, noise, k):
    del k
    lgT = logits.T
    nzT = noise.T
    idx16, mn, mx = _topk_call(lgT, nzT)
    idxk = idx16[:, :K]

    flat_idx = idxk.reshape(-1)
    idx_pad = jnp.pad(flat_idx, (0, TPAD - B * K)).reshape(NW, CPW)
    g = _sc_call(lgT, idx_pad)
    att2 = _att_call(g, mn, mx)
    att = att2.reshape(-1)[:B * K].reshape(B, K)
    return att, idxk
